# baseline (speedup 1.0000x reference)
import functools

import jax
import jax.numpy as jnp
from jax.experimental import pallas as pl
from jax.experimental.pallas import tpu as pltpu
from jax.experimental.pallas import tpu_sc as plsc

N = 10000
C = 16
F = 256
H = 256
O = 256
GRID = 3
ORDER = 2
NB = GRID + ORDER
NK = GRID + 2 * ORDER + 1

NSC = 4000
TN = 1000
SC_BN = 8


def _dot_t(a, w):
    return jax.lax.dot_general(
        a, w, (((1,), (1,)), ((), ())), preferred_element_type=jnp.float32
    )


def _dense_chain(cr, am, x, wn_ref, bn_ref, wc_ref, bc_ref, wu_ref, bu_ref,
                 wb_ref, wsp_ref, grid_ref):
    h = _dot_t(x, wn_ref[...]) + bn_ref[...][None, :]
    h = h + _dot_t(cr, wc_ref[...]) + am * bc_ref[...][None, :]
    u = _dot_t(h, wu_ref[...]) + bu_ref[...][None, :]

    base = _dot_t(u * jax.nn.sigmoid(u), wb_ref[...])

    t0 = grid_ref[0:1, 0:1]
    h_inv = 1.0 / (grid_ref[0:1, 1:2] - t0)
    s = (u - t0) * h_inv
    p = []
    for j in range(NK):
        r = jnp.maximum(s - float(j), 0.0)
        p.append(r * r)
    acc = base
    for j in range(NB):
        bj = 0.5 * ((p[j] - p[j + 3]) - 3.0 * (p[j + 1] - p[j + 2]))
        acc = acc + _dot_t(bj, wsp_ref[j])
    return acc


def _main_kernel(cm_ref, aw_ref, x_ref, wn_ref, bn_ref, wc_ref, bc_ref,
                 wu_ref, bu_ref, wb_ref, wsp_ref, grid_ref, out_ref):
    aw3 = aw_ref[...][:, :, None]
    w3 = cm_ref[...] * aw3
    h8 = w3[:, 0:8, :] + w3[:, 8:16, :]
    cr = jnp.sum(h8, axis=1) * (1.0 / C)
    aw = aw_ref[...] * (1.0 / C)
    am = jnp.sum(aw, axis=1, keepdims=True)
    out_ref[...] = _dense_chain(cr, am, x_ref[...], wn_ref, bn_ref, wc_ref,
                                bc_ref, wu_ref, bu_ref, wb_ref, wsp_ref,
                                grid_ref)


def _tail_kernel(cr_ref, aw_ref, x_ref, wn_ref, bn_ref, wc_ref, bc_ref,
                 wu_ref, bu_ref, wb_ref, wsp_ref, grid_ref, out_ref):
    aw = aw_ref[...] * (1.0 / C)
    am = jnp.sum(aw, axis=1, keepdims=True)
    out_ref[...] = _dense_chain(cr_ref[...], am, x_ref[...], wn_ref, bn_ref,
                                wc_ref, bc_ref, wu_ref, bu_ref, wb_ref,
                                wsp_ref, grid_ref)


def _sc_reduce(cm, aw):
    mesh = plsc.VectorSubcoreMesh(core_axis_name="core",
                                  subcore_axis_name="subcore")

    @functools.partial(
        pl.kernel,
        out_type=jax.ShapeDtypeStruct((NSC, F), jnp.float32),
        mesh=mesh,
    )
    def k(cm_hbm, aw_hbm, out_hbm):
        def body(cm_v, aw_v, out_v):
            @pl.loop(0, SC_BN)
            def _node(i):
                awrow = aw_v[i, :] * (1.0 / C)
                aws = [awrow[c] for c in range(C)]
                @pl.loop(0, F, step=16)
                def _feat(f):
                    sl = pl.ds(f, 16)
                    acc = aws[0] * cm_v[i, 0, sl]
                    for c in range(1, C):
                        acc = acc + aws[c] * cm_v[i, c, sl]
                    out_v[i, sl] = acc

        pltpu.emit_pipeline(
            body,
            grid=(NSC // SC_BN,),
            in_specs=[
                pl.BlockSpec((SC_BN, C, F), lambda i: (i, 0, 0)),
                pl.BlockSpec((SC_BN, C), lambda i: (i, 0)),
            ],
            out_specs=[pl.BlockSpec((SC_BN, F), lambda i: (i, 0))],
            core_axis_name=("core", "subcore"),
            dimension_semantics=(pltpu.PARALLEL,),
        )(cm_hbm, aw_hbm, out_hbm)

    return k(cm, aw)


@functools.partial(jax.jit, static_argnames=())
def kernel(x, context_map, attention_weights_map, W_node, b_node, W_ctx,
           b_ctx, W_upd, b_upd, kan_base_w, kan_spline_w, kan_grid):
    wsp = jnp.transpose(kan_spline_w, (2, 0, 1))

    full = lambda *s: pl.BlockSpec(s, lambda i: (0,) * len(s))
    w_specs = [full(H, F), full(H), full(H, F), full(H),
               full(O, H), full(O), full(O, O), full(NB, O, O),
               full(O, NK)]
    weights = (W_node, b_node, W_ctx, b_ctx, W_upd, b_upd, kan_base_w, wsp,
               kan_grid)

    cr_sc = _sc_reduce(context_map, attention_weights_map)

    k0 = NSC // TN
    out_main = pl.pallas_call(
        _main_kernel,
        grid=((N - NSC) // TN,),
        in_specs=[
            pl.BlockSpec((TN, C, F), lambda i: (i + k0, 0, 0)),
            pl.BlockSpec((TN, C), lambda i: (i + k0, 0)),
            pl.BlockSpec((TN, F), lambda i: (i + k0, 0)),
            *w_specs,
        ],
        out_specs=pl.BlockSpec((TN, O), lambda i: (i, 0)),
        out_shape=jax.ShapeDtypeStruct((N - NSC, O), jnp.float32),
    )(context_map, attention_weights_map, x, *weights)

    out_tail = pl.pallas_call(
        _tail_kernel,
        grid=(NSC // TN,),
        in_specs=[
            pl.BlockSpec((TN, F), lambda i: (i, 0)),
            pl.BlockSpec((TN, C), lambda i: (i, 0)),
            pl.BlockSpec((TN, F), lambda i: (i, 0)),
            *w_specs,
        ],
        out_specs=pl.BlockSpec((TN, O), lambda i: (i, 0)),
        out_shape=jax.ShapeDtypeStruct((NSC, O), jnp.float32),
    )(cr_sc, attention_weights_map, x, *weights)

    return jnp.concatenate([out_tail, out_main], axis=0)


# speedup vs baseline: 1.4654x; 1.4654x over previous
import functools

import jax
import jax.numpy as jnp
from jax.experimental import pallas as pl

N = 10000
C = 16
F = 256
H = 256
O = 256
GRID = 3
ORDER = 2
NB = GRID + ORDER
NK = GRID + 2 * ORDER + 1


def _dot_t(a, w):
    return jax.lax.dot_general(
        a, w, (((1,), (1,)), ((), ())), preferred_element_type=jnp.float32
    )


def _fused_kernel(cm_ref, aw_ref, x_ref, wn_ref, bn_ref, wc_ref,
                  bc_ref, wu_ref, bu_ref, wb_ref, wsp_ref, grid_ref, out_ref):
    aw3 = aw_ref[...][:, :, None]
    w3 = cm_ref[...] * aw3
    h8 = w3[:, 0:8, :] + w3[:, 8:16, :]
    cr = jnp.sum(h8, axis=1) * (1.0 / C)
    aw = aw_ref[...] * (1.0 / C)
    am = jnp.sum(aw, axis=1, keepdims=True)

    h = _dot_t(x_ref[...], wn_ref[...]) + bn_ref[...][None, :]
    h = h + _dot_t(cr, wc_ref[...]) + am * bc_ref[...][None, :]
    u = _dot_t(h, wu_ref[...]) + bu_ref[...][None, :]

    base = _dot_t(u * jax.nn.sigmoid(u), wb_ref[...])

    t0 = grid_ref[0:1, 0:1]
    h_inv = 1.0 / (grid_ref[0:1, 1:2] - t0)
    s = (u - t0) * h_inv
    p = []
    for j in range(NK):
        r = jnp.maximum(s - float(j), 0.0)
        p.append(r * r)
    acc = base
    for j in range(NB):
        bj = 0.5 * ((p[j] - p[j + 3]) - 3.0 * (p[j + 1] - p[j + 2]))
        acc = acc + _dot_t(bj, wsp_ref[j])
    out_ref[...] = acc


@functools.partial(jax.jit, static_argnames=())
def kernel(x, context_map, attention_weights_map, W_node, b_node, W_ctx,
           b_ctx, W_upd, b_upd, kan_base_w, kan_spline_w, kan_grid):
    Tn = 1000
    grid = (N // Tn,)
    wsp = jnp.transpose(kan_spline_w, (2, 0, 1))

    full = lambda *s: pl.BlockSpec(s, lambda i: (0,) * len(s))
    return pl.pallas_call(
        _fused_kernel,
        grid=grid,
        in_specs=[
            pl.BlockSpec((Tn, C, F), lambda i: (i, 0, 0)),
            pl.BlockSpec((Tn, C), lambda i: (i, 0)),
            pl.BlockSpec((Tn, F), lambda i: (i, 0)),
            full(H, F), full(H), full(H, F), full(H),
            full(O, H), full(O), full(O, O), full(NB, O, O),
            full(O, NK),
        ],
        out_specs=pl.BlockSpec((Tn, O), lambda i: (i, 0)),
        out_shape=jax.ShapeDtypeStruct((N, O), jnp.float32),
    )(context_map, attention_weights_map, x, W_node, b_node, W_ctx,
      b_ctx, W_upd, b_upd, kan_base_w, wsp, kan_grid)
